# baseline (speedup 1.0000x reference)
import jax
import jax.numpy as jnp
from jax import lax
from jax.experimental import pallas as pl
from jax.experimental.pallas import tpu as pltpu
from jax.experimental.pallas import tpu_sc as plsc

B, S, H = 128, 512, 768
EPS = 1e-12
NC, NS, L = 2, 16, 16
NW = NC * NS
SBLK = S // NW
INV_H = 1.0 / H
NBUF = 2
NYBUF = 3


def _rsqrt_vec(t):
    i = lax.bitcast_convert_type(t, jnp.int32)
    i = jnp.int32(0x5F3759DF) - lax.shift_right_logical(i, 1)
    y = lax.bitcast_convert_type(i, jnp.float32)
    for _ in range(4):
        y = y * (1.5 - 0.5 * t * y * y)
    return y


def _body(ids_hbm, wemb_hbm, pos_hbm, type_hbm, gamma_hbm, beta_hbm,
          out_hbm, idx_v, rows0_v, rows1_v, y0_v, y1_v, y2_v,
          pe_v, ty_v, g_v, b_v,
          gsem0, gsem1, ssem0, ssem1, ssem2):
    wid = lax.axis_index("s") * NC + lax.axis_index("c")
    sbase = wid * SBLK

    pltpu.sync_copy(ids_hbm.at[:, pl.ds(sbase, SBLK)], idx_v)
    pltpu.sync_copy(pos_hbm.at[pl.ds(sbase, SBLK)], pe_v)
    pltpu.sync_copy(type_hbm.at[pl.ds(0, 1)], ty_v)
    pltpu.sync_copy(gamma_hbm, g_v)
    pltpu.sync_copy(beta_hbm, b_v)

    @plsc.parallel_loop(0, SBLK)
    def fold_type(r):
        @plsc.parallel_loop(0, H, step=L, unroll=4)
        def fchunk(c):
            pe_v[r, pl.ds(c, L)] = pe_v[r, pl.ds(c, L)] + ty_v[0, pl.ds(c, L)]

    zeros = jnp.zeros((L,), jnp.float32)
    rows_bufs = ((rows0_v, gsem0), (rows1_v, gsem1))
    y_bufs = ((y0_v, ssem0), (y1_v, ssem1), (y2_v, ssem2))

    def gather_start(b, rows_v, gsem):
        pltpu.make_async_copy(wemb_hbm.at[idx_v.at[b]], rows_v, gsem).start()

    for p in range(NBUF):
        gather_start(p, *rows_bufs[p])

    def step(b, rows_v, gsem, y_v, ssem):
        pltpu.make_async_copy(wemb_hbm.at[idx_v.at[b]], rows_v, gsem).wait()

        @pl.when(b >= NYBUF)
        def _():
            pltpu.make_async_copy(
                y_v, out_hbm.at[pl.ds((b - NYBUF) * S + sbase, SBLK)], ssem
            ).wait()

        def normalize(r, mean, rinv):
            @plsc.parallel_loop(0, H, step=L, unroll=8)
            def pass2(c):
                x = y_v[r, pl.ds(c, L)]
                y = (x - mean) * rinv * g_v[pl.ds(c, L)] + b_v[pl.ds(c, L)]
                y_v[r, pl.ds(c, L)] = y

        @plsc.parallel_loop(0, SBLK, carry=(zeros, zeros))
        def row_body(r, stats):
            @plsc.parallel_loop(0, H, step=L, unroll=8, carry=(zeros, zeros))
            def moments(c, carry):
                acc, acc2 = carry
                x = rows_v[r, pl.ds(c, L)] + pe_v[r, pl.ds(c, L)]
                y_v[r, pl.ds(c, L)] = x
                return acc + x, acc2 + x * x

            acc, acc2 = moments
            s1 = jnp.sum(acc)
            s2 = jnp.sum(acc2)
            mean = s1 * INV_H
            var = jnp.maximum(s2 * INV_H - mean * mean, 0.0)
            rinv = _rsqrt_vec(jnp.full((L,), var + EPS, jnp.float32))

            pmean, prinv = stats

            @pl.when(r >= 1)
            def _():
                normalize(r - 1, pmean, prinv)

            return jnp.full((L,), mean, jnp.float32), rinv

        lmean, lrinv = row_body
        normalize(SBLK - 1, lmean, lrinv)

        pltpu.make_async_copy(
            y_v, out_hbm.at[pl.ds(b * S + sbase, SBLK)], ssem
        ).start()

        @pl.when(b + NBUF < B)
        def _():
            gather_start(b + NBUF, rows_v, gsem)

    def loop_body(i, _):
        b = i * 6
        for p in range(6):
            bb = b + p
            step(bb, *rows_bufs[p % NBUF], *y_bufs[p % NYBUF])
        return 0

    lax.fori_loop(0, B // 6, loop_body, 0)
    step(126, *rows_bufs[0], *y_bufs[0])
    step(127, *rows_bufs[1], *y_bufs[1])

    for bb in range(B - NYBUF, B):
        y_v, ssem = y_bufs[bb % NYBUF]
        pltpu.make_async_copy(
            y_v, out_hbm.at[pl.ds(bb * S + sbase, SBLK)], ssem).wait()


@jax.jit
def kernel(input_ids, word_emb, pos_emb, type_emb, ln_gamma, ln_beta):
    ids = input_ids.astype(jnp.int32)
    mesh = plsc.VectorSubcoreMesh(core_axis_name="c", subcore_axis_name="s")
    out = pl.kernel(
        _body,
        mesh=mesh,
        compiler_params=pltpu.CompilerParams(
            use_tc_tiling_on_sc=False, needs_layout_passes=False),
        out_type=jax.ShapeDtypeStruct((B * S, H), jnp.float32),
        scratch_types=[
            pltpu.VMEM((B, SBLK), jnp.int32),
            pltpu.VMEM((SBLK, H), jnp.float32),
            pltpu.VMEM((SBLK, H), jnp.float32),
            pltpu.VMEM((SBLK, H), jnp.float32),
            pltpu.VMEM((SBLK, H), jnp.float32),
            pltpu.VMEM((SBLK, H), jnp.float32),
            pltpu.VMEM((SBLK, H), jnp.float32),
            pltpu.VMEM((1, H), jnp.float32),
            pltpu.VMEM((H,), jnp.float32),
            pltpu.VMEM((H,), jnp.float32),
            pltpu.SemaphoreType.DMA,
            pltpu.SemaphoreType.DMA,
            pltpu.SemaphoreType.DMA,
            pltpu.SemaphoreType.DMA,
            pltpu.SemaphoreType.DMA,
        ],
    )(ids, word_emb, pos_emb, type_emb, ln_gamma, ln_beta)
    return out.reshape(B, S, H)


# speedup vs baseline: 1.0414x; 1.0414x over previous
import jax
import jax.numpy as jnp
from jax import lax
from jax.experimental import pallas as pl
from jax.experimental.pallas import tpu as pltpu
from jax.experimental.pallas import tpu_sc as plsc

B, S, H = 128, 512, 768
EPS = 1e-12
NC, NS, L = 2, 16, 16
NW = NC * NS
SBLK = S // NW
INV_H = 1.0 / H
NBUF = 2
NYBUF = 3


def _rsqrt_vec(t):
    i = lax.bitcast_convert_type(t, jnp.int32)
    i = jnp.int32(0x5F3759DF) - lax.shift_right_logical(i, 1)
    y = lax.bitcast_convert_type(i, jnp.float32)
    for _ in range(4):
        y = y * (1.5 - 0.5 * t * y * y)
    return y


def _body(ids_hbm, wemb_hbm, pos_hbm, type_hbm, gamma_hbm, beta_hbm,
          out_hbm, idx_v, rows0_v, rows1_v, y0_v, y1_v, y2_v,
          pe_v, ty_v, g_v, b_v,
          gsem0, gsem1, ssem0, ssem1, ssem2):
    wid = lax.axis_index("s") * NC + lax.axis_index("c")
    sbase = wid * SBLK

    pltpu.sync_copy(ids_hbm.at[:, pl.ds(sbase, SBLK)], idx_v)
    pltpu.sync_copy(pos_hbm.at[pl.ds(sbase, SBLK)], pe_v)
    pltpu.sync_copy(type_hbm.at[pl.ds(0, 1)], ty_v)
    pltpu.sync_copy(gamma_hbm, g_v)
    pltpu.sync_copy(beta_hbm, b_v)

    @plsc.parallel_loop(0, SBLK)
    def fold_type(r):
        @plsc.parallel_loop(0, H, step=L, unroll=4)
        def fchunk(c):
            pe_v[r, pl.ds(c, L)] = pe_v[r, pl.ds(c, L)] + ty_v[0, pl.ds(c, L)]

    zeros = jnp.zeros((L,), jnp.float32)
    rows_bufs = ((rows0_v, gsem0), (rows1_v, gsem1))
    y_bufs = ((y0_v, ssem0), (y1_v, ssem1), (y2_v, ssem2))

    def gather_start(b, rows_v, gsem):
        pltpu.make_async_copy(wemb_hbm.at[idx_v.at[b]], rows_v, gsem).start()

    for p in range(NBUF):
        gather_start(p, *rows_bufs[p])

    def step(b, rows_v, gsem, y_v, ssem):
        pltpu.make_async_copy(wemb_hbm.at[idx_v.at[b]], rows_v, gsem).wait()

        @pl.when(b >= NYBUF)
        def _():
            pltpu.make_async_copy(
                y_v, out_hbm.at[pl.ds((b - NYBUF) * S + sbase, SBLK)], ssem
            ).wait()

        @plsc.parallel_loop(0, SBLK)
        def row_body(r):
            @plsc.parallel_loop(0, H, step=L, unroll=16, carry=(zeros, zeros))
            def moments(c, carry):
                acc, acc2 = carry
                x = rows_v[r, pl.ds(c, L)] + pe_v[r, pl.ds(c, L)]
                y_v[r, pl.ds(c, L)] = x
                return acc + x, acc2 + x * x

            acc, acc2 = moments
            s1 = jnp.sum(acc)
            s2 = jnp.sum(acc2)
            mean = s1 * INV_H
            var = jnp.maximum(s2 * INV_H - mean * mean, 0.0)
            rinv = _rsqrt_vec(jnp.full((L,), var + EPS, jnp.float32))

            @plsc.parallel_loop(0, H, step=L, unroll=16)
            def pass2(c):
                x = y_v[r, pl.ds(c, L)]
                y = (x - mean) * rinv * g_v[pl.ds(c, L)] + b_v[pl.ds(c, L)]
                y_v[r, pl.ds(c, L)] = y

        pltpu.make_async_copy(
            y_v, out_hbm.at[pl.ds(b * S + sbase, SBLK)], ssem
        ).start()

        @pl.when(b + NBUF < B)
        def _():
            gather_start(b + NBUF, rows_v, gsem)

    def loop_body(i, _):
        b = i * 6
        for p in range(6):
            bb = b + p
            step(bb, *rows_bufs[p % NBUF], *y_bufs[p % NYBUF])
        return 0

    lax.fori_loop(0, B // 6, loop_body, 0)
    step(126, *rows_bufs[0], *y_bufs[0])
    step(127, *rows_bufs[1], *y_bufs[1])

    for bb in range(B - NYBUF, B):
        y_v, ssem = y_bufs[bb % NYBUF]
        pltpu.make_async_copy(
            y_v, out_hbm.at[pl.ds(bb * S + sbase, SBLK)], ssem).wait()


@jax.jit
def kernel(input_ids, word_emb, pos_emb, type_emb, ln_gamma, ln_beta):
    ids = input_ids.astype(jnp.int32)
    mesh = plsc.VectorSubcoreMesh(core_axis_name="c", subcore_axis_name="s")
    out = pl.kernel(
        _body,
        mesh=mesh,
        compiler_params=pltpu.CompilerParams(
            use_tc_tiling_on_sc=False, needs_layout_passes=False),
        out_type=jax.ShapeDtypeStruct((B * S, H), jnp.float32),
        scratch_types=[
            pltpu.VMEM((B, SBLK), jnp.int32),
            pltpu.VMEM((SBLK, H), jnp.float32),
            pltpu.VMEM((SBLK, H), jnp.float32),
            pltpu.VMEM((SBLK, H), jnp.float32),
            pltpu.VMEM((SBLK, H), jnp.float32),
            pltpu.VMEM((SBLK, H), jnp.float32),
            pltpu.VMEM((SBLK, H), jnp.float32),
            pltpu.VMEM((1, H), jnp.float32),
            pltpu.VMEM((H,), jnp.float32),
            pltpu.VMEM((H,), jnp.float32),
            pltpu.SemaphoreType.DMA,
            pltpu.SemaphoreType.DMA,
            pltpu.SemaphoreType.DMA,
            pltpu.SemaphoreType.DMA,
            pltpu.SemaphoreType.DMA,
        ],
    )(ids, word_emb, pos_emb, type_emb, ln_gamma, ln_beta)
    return out.reshape(B, S, H)


# speedup vs baseline: 1.0497x; 1.0081x over previous
import jax
import jax.numpy as jnp
from jax import lax
from jax.experimental import pallas as pl
from jax.experimental.pallas import tpu as pltpu
from jax.experimental.pallas import tpu_sc as plsc

B, S, H = 128, 512, 768
EPS = 1e-12
NC, NS, L = 2, 16, 16
NW = NC * NS
SBLK = S // NW
INV_H = 1.0 / H
NBUF = 2
NYBUF = 3


def _rsqrt_vec(t):
    i = lax.bitcast_convert_type(t, jnp.int32)
    i = jnp.int32(0x5F3759DF) - lax.shift_right_logical(i, 1)
    y = lax.bitcast_convert_type(i, jnp.float32)
    for _ in range(4):
        y = y * (1.5 - 0.5 * t * y * y)
    return y


def _body(ids_hbm, wemb_hbm, pos_hbm, type_hbm, gamma_hbm, beta_hbm,
          out_hbm, idx_v, rows0_v, rows1_v, y0_v, y1_v, y2_v,
          pe_v, ty_v, g_v, b_v,
          gsem0, gsem1, ssem0, ssem1, ssem2):
    wid = lax.axis_index("s") * NC + lax.axis_index("c")
    sbase = wid * SBLK

    pltpu.sync_copy(ids_hbm.at[:, pl.ds(sbase, SBLK)], idx_v)
    pltpu.sync_copy(pos_hbm.at[pl.ds(sbase, SBLK)], pe_v)
    pltpu.sync_copy(type_hbm.at[pl.ds(0, 1)], ty_v)
    pltpu.sync_copy(gamma_hbm, g_v)
    pltpu.sync_copy(beta_hbm, b_v)

    @plsc.parallel_loop(0, SBLK)
    def fold_type(r):
        @plsc.parallel_loop(0, H, step=L, unroll=4)
        def fchunk(c):
            pe_v[r, pl.ds(c, L)] = pe_v[r, pl.ds(c, L)] + ty_v[0, pl.ds(c, L)]

    zeros = jnp.zeros((L,), jnp.float32)
    rows_bufs = ((rows0_v, gsem0), (rows1_v, gsem1))
    y_bufs = ((y0_v, ssem0), (y1_v, ssem1), (y2_v, ssem2))

    def gather_start(b, rows_v, gsem):
        pltpu.make_async_copy(wemb_hbm.at[idx_v.at[b]], rows_v, gsem).start()

    for p in range(NBUF):
        gather_start(p, *rows_bufs[p])

    def step(b, rows_v, gsem, y_v, ssem):
        pltpu.make_async_copy(wemb_hbm.at[idx_v.at[b]], rows_v, gsem).wait()

        @pl.when(b >= NYBUF)
        def _():
            pltpu.make_async_copy(
                y_v, out_hbm.at[pl.ds((b - NYBUF) * S + sbase, SBLK)], ssem
            ).wait()

        @plsc.parallel_loop(0, SBLK)
        def row_body(r):
            @plsc.parallel_loop(0, H, step=L, unroll=24, carry=(zeros, zeros))
            def moments(c, carry):
                acc, acc2 = carry
                x = rows_v[r, pl.ds(c, L)] + pe_v[r, pl.ds(c, L)]
                y_v[r, pl.ds(c, L)] = x
                return acc + x, acc2 + x * x

            acc, acc2 = moments
            s1 = jnp.sum(acc)
            s2 = jnp.sum(acc2)
            mean = s1 * INV_H
            var = jnp.maximum(s2 * INV_H - mean * mean, 0.0)
            rinv = _rsqrt_vec(jnp.full((L,), var + EPS, jnp.float32))

            @plsc.parallel_loop(0, H, step=L, unroll=24)
            def pass2(c):
                x = y_v[r, pl.ds(c, L)]
                y = (x - mean) * rinv * g_v[pl.ds(c, L)] + b_v[pl.ds(c, L)]
                y_v[r, pl.ds(c, L)] = y

        pltpu.make_async_copy(
            y_v, out_hbm.at[pl.ds(b * S + sbase, SBLK)], ssem
        ).start()

        @pl.when(b + NBUF < B)
        def _():
            gather_start(b + NBUF, rows_v, gsem)

    def loop_body(i, _):
        b = i * 6
        for p in range(6):
            bb = b + p
            step(bb, *rows_bufs[p % NBUF], *y_bufs[p % NYBUF])
        return 0

    lax.fori_loop(0, B // 6, loop_body, 0)
    step(126, *rows_bufs[0], *y_bufs[0])
    step(127, *rows_bufs[1], *y_bufs[1])

    for bb in range(B - NYBUF, B):
        y_v, ssem = y_bufs[bb % NYBUF]
        pltpu.make_async_copy(
            y_v, out_hbm.at[pl.ds(bb * S + sbase, SBLK)], ssem).wait()


@jax.jit
def kernel(input_ids, word_emb, pos_emb, type_emb, ln_gamma, ln_beta):
    ids = input_ids.astype(jnp.int32)
    mesh = plsc.VectorSubcoreMesh(core_axis_name="c", subcore_axis_name="s")
    out = pl.kernel(
        _body,
        mesh=mesh,
        compiler_params=pltpu.CompilerParams(
            use_tc_tiling_on_sc=False, needs_layout_passes=False),
        out_type=jax.ShapeDtypeStruct((B * S, H), jnp.float32),
        scratch_types=[
            pltpu.VMEM((B, SBLK), jnp.int32),
            pltpu.VMEM((SBLK, H), jnp.float32),
            pltpu.VMEM((SBLK, H), jnp.float32),
            pltpu.VMEM((SBLK, H), jnp.float32),
            pltpu.VMEM((SBLK, H), jnp.float32),
            pltpu.VMEM((SBLK, H), jnp.float32),
            pltpu.VMEM((SBLK, H), jnp.float32),
            pltpu.VMEM((1, H), jnp.float32),
            pltpu.VMEM((H,), jnp.float32),
            pltpu.VMEM((H,), jnp.float32),
            pltpu.SemaphoreType.DMA,
            pltpu.SemaphoreType.DMA,
            pltpu.SemaphoreType.DMA,
            pltpu.SemaphoreType.DMA,
            pltpu.SemaphoreType.DMA,
        ],
    )(ids, word_emb, pos_emb, type_emb, ln_gamma, ln_beta)
    return out.reshape(B, S, H)
